# baseline (speedup 1.0000x reference)
import functools

import jax
import jax.numpy as jnp
from jax.experimental import pallas as pl
from jax.experimental.pallas import tpu as pltpu


def _se_body(x_ref, w_ref, b_ref, o_ref, *, inv_hw):
    x = x_ref[...]
    pooled = jnp.sum(x, axis=(1, 2)) * inv_hw
    logits = jax.lax.dot_general(
        pooled, w_ref[...], (((1,), (1,)), ((), ())),
        preferred_element_type=jnp.float32,
    ) + b_ref[...]
    gate = jax.nn.sigmoid(logits)
    o_ref[...] = x * gate[:, None, None, :]


def kernel(x, weight, bias):
    B, C, H, W = x.shape
    bblk = 2 if B % 2 == 0 else 1
    x_nhwc = jnp.transpose(x, (0, 2, 3, 1))
    w = jnp.asarray(weight).reshape(C, C)
    b_row = jnp.asarray(bias).reshape(1, C)

    out = pl.pallas_call(
        functools.partial(_se_body, inv_hw=1.0 / (H * W)),
        out_shape=jax.ShapeDtypeStruct((B, H, W, C), x.dtype),
        grid=(B // bblk,),
        in_specs=[
            pl.BlockSpec((bblk, H, W, C), lambda b: (b, 0, 0, 0)),
            pl.BlockSpec((C, C), lambda b: (0, 0)),
            pl.BlockSpec((1, C), lambda b: (0, 0)),
        ],
        out_specs=pl.BlockSpec((bblk, H, W, C), lambda b: (b, 0, 0, 0)),
        compiler_params=pltpu.CompilerParams(
            dimension_semantics=("parallel",),
            vmem_limit_bytes=56 << 20,
        ),
    )(x_nhwc, w, b_row)
    return jnp.transpose(out, (0, 3, 1, 2))


# speedup vs baseline: 1.2263x; 1.2263x over previous
import functools

import jax
import jax.numpy as jnp
from jax.experimental import pallas as pl
from jax.experimental.pallas import tpu as pltpu


def _se_body(x_ref, w_ref, b_ref, o_ref, *, inv_hw):
    x = x_ref[...]
    pooled = jnp.sum(x, axis=(1, 2)) * inv_hw
    logits = jax.lax.dot_general(
        pooled, w_ref[...], (((1,), (1,)), ((), ())),
        preferred_element_type=jnp.float32,
    ) + b_ref[...]
    gate = jax.nn.sigmoid(logits)
    o_ref[...] = x * gate[:, None, None, :]


def kernel(x, weight, bias):
    B, C, H, W = x.shape
    bblk = 8 if B % 8 == 0 else 1
    x_nhwc = jnp.transpose(x, (0, 2, 3, 1))
    w = jnp.asarray(weight).reshape(C, C)
    b_row = jnp.asarray(bias).reshape(1, C)

    out = pl.pallas_call(
        functools.partial(_se_body, inv_hw=1.0 / (H * W)),
        out_shape=jax.ShapeDtypeStruct((B, H, W, C), x.dtype),
        grid=(B // bblk,),
        in_specs=[
            pl.BlockSpec((bblk, H, W, C), lambda b: (b, 0, 0, 0)),
            pl.BlockSpec((C, C), lambda b: (0, 0)),
            pl.BlockSpec((1, C), lambda b: (0, 0)),
        ],
        out_specs=pl.BlockSpec((bblk, H, W, C), lambda b: (b, 0, 0, 0)),
        compiler_params=pltpu.CompilerParams(
            dimension_semantics=("parallel",),
            vmem_limit_bytes=56 << 20,
        ),
    )(x_nhwc, w, b_row)
    return jnp.transpose(out, (0, 3, 1, 2))
